# baseline (speedup 1.0000x reference)
import functools

import jax
import jax.numpy as jnp
from jax import lax
from jax.experimental import pallas as pl
from jax.experimental.pallas import tpu as pltpu
from jax.experimental.pallas import tpu_sc as plsc

NC = 2
NS = 16
NW = NC * NS
L = 16

BATCH = 16384
DIM = 64
B_PER_W = BATCH // NW
CHUNK = 128
NCHUNK = B_PER_W // CHUNK


def _sc_body(cidx_hbm, xidx_hbm, cw_hbm, xw_hbm, out_hbm,
             cidx_v, xidx_v, crows_v, xrows_v, out_v, sem):
    wid = lax.axis_index("s") * NC + lax.axis_index("c")
    base = wid * B_PER_W

    pltpu.sync_copy(cidx_hbm.at[pl.ds(wid * NCHUNK, NCHUNK)], cidx_v)
    pltpu.sync_copy(xidx_hbm.at[pl.ds(wid * NCHUNK, NCHUNK)], xidx_v)

    copies = []
    for j in range(NCHUNK):
        copies.append(pltpu.async_copy(
            cw_hbm.at[cidx_v.at[j]], crows_v.at[pl.ds(j * CHUNK, CHUNK)], sem))
        copies.append(pltpu.async_copy(
            xw_hbm.at[xidx_v.at[j]], xrows_v.at[pl.ds(j * CHUNK, CHUNK)], sem))
    for c in copies:
        c.wait()

    iota = lax.iota(jnp.int32, L)

    def group(g, carry):
        r0 = g * L
        vec = jnp.zeros((L,), jnp.float32)
        for u in range(L):
            r = r0 + u
            s = jnp.zeros((L,), jnp.float32)
            for k in range(DIM // L):
                cg = crows_v[r, pl.ds(k * L, L)]
                xg = xrows_v[r, pl.ds(k * L, L)]
                s = s + cg * xg
            vec = jnp.where(iota == u, jnp.sum(s), vec)
        out_v[pl.ds(r0, L)] = vec
        return carry

    lax.fori_loop(0, B_PER_W // L, group, 0)

    pltpu.sync_copy(out_v, out_hbm.at[pl.ds(base, B_PER_W)])


@functools.partial(jax.jit, static_argnames=())
def _run(cidx, xidx, cw, xw):
    mesh = plsc.VectorSubcoreMesh(
        core_axis_name="c", subcore_axis_name="s",
        num_cores=NC, num_subcores=NS)
    f = pl.kernel(
        _sc_body,
        out_type=jax.ShapeDtypeStruct((BATCH,), jnp.float32),
        mesh=mesh,
        compiler_params=pltpu.CompilerParams(
            needs_layout_passes=False, use_tc_tiling_on_sc=False),
        scratch_types=[
            pltpu.VMEM((NCHUNK, CHUNK), jnp.int32),
            pltpu.VMEM((NCHUNK, CHUNK), jnp.int32),
            pltpu.VMEM((B_PER_W, DIM), jnp.float32),
            pltpu.VMEM((B_PER_W, DIM), jnp.float32),
            pltpu.VMEM((B_PER_W,), jnp.float32),
            pltpu.SemaphoreType.DMA,
        ],
    )
    return f(cidx, xidx, cw, xw)


def kernel(center_idx, context_idx, center_W, context_W):
    cidx = center_idx.astype(jnp.int32).reshape(NW * NCHUNK, CHUNK)
    xidx = context_idx.astype(jnp.int32).reshape(NW * NCHUNK, CHUNK)
    return _run(cidx, xidx, center_W, context_W)
